# speedup vs baseline: 3.6402x; 3.6402x over previous
import functools

import jax
import jax.numpy as jnp
from jax import lax
from jax.experimental import pallas as pl
from jax.experimental.pallas import tpu as pltpu
from jax.experimental.pallas import tpu_sc as plsc

_N = 10000
_D = 128
_E = 320000
_NC, _NS = 2, 16
_NW = _NC * _NS
_K = 128
_EPAD = 323584
_T = _EPAD // _NW
_CHUNKS = _T // _K
_NPAD = 10240
_ROWS_PER_TILE = _NPAD // _NS
_TS = 3
_EPS = 1e-8
_R = 1000


def _mesh():
    return plsc.VectorSubcoreMesh(
        core_axis_name="c", subcore_axis_name="s",
        num_cores=_NC, num_subcores=_NS)


@functools.partial(
    pl.kernel,
    out_type=jax.ShapeDtypeStruct((_NC, _NPAD, _D), jnp.float32),
    mesh=_mesh(),
    scratch_types=[
        pltpu.VMEM_SHARED((_NPAD, _D), jnp.float32),
        pltpu.VMEM((_K,), jnp.int32),
        pltpu.VMEM((_K,), jnp.int32),
        pltpu.VMEM((_K, _D), jnp.float32),
        pltpu.VMEM((_K, _D), jnp.float32),
        pltpu.SemaphoreType.DMA,
    ],
)
def _sc_scatter(prop, srcr, dstr, zeros_hbm, out, acc, idx_s, idx_d, rows,
                zbuf, sem):
    c = lax.axis_index("c")
    s = lax.axis_index("s")
    pltpu.sync_copy(zeros_hbm, zbuf)
    for z in range(_ROWS_PER_TILE // _K):
        pltpu.sync_copy(zbuf, acc.at[pl.ds((s * (_ROWS_PER_TILE // _K) + z) * _K, _K)])
    plsc.subcore_barrier()
    base = (c * _NS + s) * _T

    def step(j, carry):
        off = pl.multiple_of(base + j * _K, 8)
        pltpu.sync_copy(srcr.at[pl.ds(off, _K)], idx_s)
        pltpu.sync_copy(dstr.at[pl.ds(off, _K)], idx_d)
        pltpu.async_copy(prop.at[idx_s], rows, sem).wait()
        pltpu.sync_copy(rows, acc.at[idx_d], add=True)
        return carry

    lax.fori_loop(0, _CHUNKS, step, 0)
    plsc.subcore_barrier()
    pltpu.sync_copy(acc.at[pl.ds(s * _ROWS_PER_TILE, _ROWS_PER_TILE)],
                    out.at[c, pl.ds(s * _ROWS_PER_TILE, _ROWS_PER_TILE)])


@functools.partial(
    pl.kernel,
    out_type=jax.ShapeDtypeStruct((_NC, _NPAD, _D), jnp.float32),
    mesh=_mesh(),
    scratch_types=[
        pltpu.VMEM_SHARED((_NPAD, _D), jnp.float32),
        pltpu.VMEM((_K,), jnp.int32),
        pltpu.VMEM((_K, _D), jnp.float32),
        pltpu.VMEM((_K, _D), jnp.float32),
    ],
)
def _sc_count(dstr, col0_hbm, zeros_hbm, out, acc, idx_d, ones, zbuf):
    c = lax.axis_index("c")
    s = lax.axis_index("s")
    pltpu.sync_copy(zeros_hbm, zbuf)
    for z in range(_ROWS_PER_TILE // _K):
        pltpu.sync_copy(zbuf, acc.at[pl.ds((s * (_ROWS_PER_TILE // _K) + z) * _K, _K)])
    pltpu.sync_copy(col0_hbm, ones)
    plsc.subcore_barrier()
    base = (c * _NS + s) * _T

    def step(j, carry):
        off = pl.multiple_of(base + j * _K, 8)
        pltpu.sync_copy(dstr.at[pl.ds(off, _K)], idx_d)
        pltpu.sync_copy(ones, acc.at[idx_d], add=True)
        return carry

    lax.fori_loop(0, _CHUNKS, step, 0)
    plsc.subcore_barrier()
    pltpu.sync_copy(acc.at[pl.ds(s * _ROWS_PER_TILE, _ROWS_PER_TILE)],
                    out.at[c, pl.ds(s * _ROWS_PER_TILE, _ROWS_PER_TILE)])


def _prop_body(h_ref, wt_ref, b_ref, o_ref):
    o_ref[...] = (jnp.dot(h_ref[...], wt_ref[...],
                          preferred_element_type=jnp.float32) + b_ref[...])


def _prop_call(h, wt, b2):
    return pl.pallas_call(
        _prop_body,
        grid=(_N // _R,),
        in_specs=[
            pl.BlockSpec((_R, _D), lambda i: (i, 0)),
            pl.BlockSpec((_D, _D), lambda i: (0, 0)),
            pl.BlockSpec((1, _D), lambda i: (0, 0)),
        ],
        out_specs=pl.BlockSpec((_R, _D), lambda i: (i, 0)),
        out_shape=jax.ShapeDtypeStruct((_N, _D), jnp.float32),
    )(h, wt, b2)


def _gru_body(msg_ref, cnt_ref, h_ref, wih_ref, whh_ref, bih_ref, bhh_ref,
              wt_ref, b_ref, hn_ref, prop_ref):
    msum = msg_ref[0] + msg_ref[1]
    cnt = cnt_ref[0, :, :1] + cnt_ref[1, :, :1]
    div = jnp.where(cnt == 0.0, 1.0, cnt) + _EPS
    x = msum / div
    h = h_ref[...]
    gi = jnp.dot(x, wih_ref[...], preferred_element_type=jnp.float32) + bih_ref[...]
    gh = jnp.dot(h, whh_ref[...], preferred_element_type=jnp.float32) + bhh_ref[...]
    r = jax.nn.sigmoid(gi[:, :_D] + gh[:, :_D])
    z = jax.nn.sigmoid(gi[:, _D:2 * _D] + gh[:, _D:2 * _D])
    n = jnp.tanh(gi[:, 2 * _D:] + r * gh[:, 2 * _D:])
    hn = (1.0 - z) * n + z * h
    hn_ref[...] = hn
    prop_ref[...] = (jnp.dot(hn, wt_ref[...],
                             preferred_element_type=jnp.float32) + b_ref[...])


def _gru_call(msg2, cnt2, h, wih_t, whh_t, bih2, bhh2, wt, b2):
    return pl.pallas_call(
        _gru_body,
        grid=(_N // _R,),
        in_specs=[
            pl.BlockSpec((_NC, _R, _D), lambda i: (0, i, 0)),
            pl.BlockSpec((_NC, _R, _D), lambda i: (0, i, 0)),
            pl.BlockSpec((_R, _D), lambda i: (i, 0)),
            pl.BlockSpec((_D, 3 * _D), lambda i: (0, 0)),
            pl.BlockSpec((_D, 3 * _D), lambda i: (0, 0)),
            pl.BlockSpec((1, 3 * _D), lambda i: (0, 0)),
            pl.BlockSpec((1, 3 * _D), lambda i: (0, 0)),
            pl.BlockSpec((_D, _D), lambda i: (0, 0)),
            pl.BlockSpec((1, _D), lambda i: (0, 0)),
        ],
        out_specs=[
            pl.BlockSpec((_R, _D), lambda i: (i, 0)),
            pl.BlockSpec((_R, _D), lambda i: (i, 0)),
        ],
        out_shape=[
            jax.ShapeDtypeStruct((_N, _D), jnp.float32),
            jax.ShapeDtypeStruct((_N, _D), jnp.float32),
        ],
    )(msg2, cnt2, h, wih_t, whh_t, bih2, bhh2, wt, b2)


def kernel(node_states, edge_lists, pos_lists, W, b, W_ih, W_hh, b_ih, b_hh):
    h = node_states
    el = edge_lists[0]
    src = el[:, 0]
    dst = el[:, 1]
    pad = _EPAD - _E
    src_p = jnp.concatenate([src, jnp.zeros((pad,), jnp.int32)])
    dst_p = jnp.concatenate([dst, jnp.full((pad,), _N, jnp.int32)])
    zeros_hbm = jnp.zeros((_K, _D), jnp.float32)
    col0 = jnp.zeros((_K, _D), jnp.float32).at[:, 0].set(1.0)
    wt = W.T
    wih_t = W_ih.T
    whh_t = W_hh.T
    bih2 = b_ih.reshape(1, -1)
    bhh2 = b_hh.reshape(1, -1)
    b2 = b.reshape(1, -1)

    cnt2 = _sc_count(dst_p, col0, zeros_hbm)
    prop = _prop_call(h, wt, b2)
    for _ in range(_TS):
        msg2 = _sc_scatter(prop, src_p, dst_p, zeros_hbm)
        h, prop = _gru_call(msg2, cnt2, h, wih_t, whh_t, bih2, bhh2, wt, b2)
    return h
